# Initial kernel scaffold; baseline (speedup 1.0000x reference)
import jax
import jax.numpy as jnp
from jax.experimental import pallas as pl


def kernel(x, y):
    raise NotImplementedError("write your pallas kernel here")


# speedup vs baseline: 6.7353x; 6.7353x over previous
import functools

import numpy as np

import jax
import jax.numpy as jnp
from jax import lax
from jax.experimental import pallas as pl
from jax.experimental.pallas import tpu as pltpu

_NUM_GAMMAS = 19


_LOG2E = 1.4426950408889634


def _exp_g(m, g):
    return jnp.exp2(m * jnp.float32(g * _LOG2E))


def _kernel_sum(m):
    acc = _exp_g(m, 1e-6)
    p2 = acc * acc
    p4 = p2 * p2
    p8 = p4 * p4
    acc = acc + p8 * p2
    for g in (1e-4, 1e-3, 1e-2, 1e-1, 1.0):
        acc = acc + _exp_g(m, g)
    e5 = _exp_g(m, 5.0)
    h = 1.0 + e5
    for _ in range(5):
        h = 1.0 + e5 * h
    acc = acc + e5 * h
    for g in (100.0, 1000.0, 1e4, 1e5, 1e6):
        acc = acc + _exp_g(m, g)
    return acc


def _mmd_task_kernel(tasks_ref, a_ref, b_ref, a2_ref, b2_ref, o_ref, *,
                     tasks_per_core, tn, n_valid, needs_mask):
    c = pl.program_id(0)
    s = pl.program_id(1)
    t = c * tasks_per_core + s

    sel_a = tasks_ref[0, t]
    ti = tasks_ref[1, t]
    sel_b = tasks_ref[2, t]
    tj = tasks_ref[3, t]
    w = tasks_ref[4, t]

    @pl.when(s == 0)
    def _init():
        o_ref[...] = jnp.zeros_like(o_ref)

    a = a_ref[...]
    b = b_ref[...]
    ab2 = lax.dot_general(a, b,
                          dimension_numbers=(((1,), (1,)), ((), ())),
                          preferred_element_type=jnp.float32)
    ssum = a2_ref[...] + b2_ref[...]
    m = jnp.minimum(ab2 - ssum, 0.0)

    is_diag = jnp.logical_and(sel_a == sel_b, ti == tj)
    rows = lax.broadcasted_iota(jnp.int32, (tn, tn), 0)
    cols = lax.broadcasted_iota(jnp.int32, (tn, tn), 1)
    m = jnp.where(jnp.logical_and(is_diag, rows == cols), jnp.float32(0.0), m)

    k = _kernel_sum(m)

    if needs_mask:
        valid = jnp.logical_and(ti * tn + rows < n_valid,
                                tj * tn + cols < n_valid)
        k = jnp.where(valid, k, jnp.float32(0.0))

    o_ref[...] += w.astype(jnp.float32) * jnp.sum(k)


def _build_tasks(nt):
    rows = []
    for sel_a, sel_b, self_k in ((0, 0, True), (1, 1, True), (0, 1, False)):
        for i in range(nt):
            for j in range(i if self_k else 0, nt):
                if self_k:
                    w = 1 if i == j else 2
                else:
                    w = -2
                rows.append((sel_a, i, sel_b, j, w))
    if len(rows) % 2:
        rows.append((0, 0, 1, 0, 0))
    return np.asarray(rows, dtype=np.int32).T


def _mmd_gaussian(x, y):
    n, d = x.shape
    scale = jnp.float32(1.0 / (_NUM_GAMMAS * n * n))

    tn = next(t for t in (1024, 512, 256, 128, 64, 32, 16, 8)
              if n % t == 0 or t == 8)
    n_pad = ((n + tn - 1) // tn) * tn
    needs_mask = n_pad != n
    if needs_mask:
        x = jnp.pad(x, ((0, n_pad - n), (0, 0)))
        y = jnp.pad(y, ((0, n_pad - n), (0, 0)))
    nt = n_pad // tn

    xh = x.astype(jnp.bfloat16)
    xl = (x - xh.astype(jnp.float32)).astype(jnp.bfloat16)
    yh = y.astype(jnp.bfloat16)
    yl = (y - yh.astype(jnp.float32)).astype(jnp.bfloat16)
    a_op = jnp.stack([jnp.concatenate([xh, xh, xl], axis=1),
                      jnp.concatenate([yh, yh, yl], axis=1)])
    b_op = 2.0 * jnp.stack([jnp.concatenate([xh, xl, xh], axis=1),
                            jnp.concatenate([yh, yl, yh], axis=1)])
    sqn = jnp.stack([jnp.sum(x * x, axis=-1), jnp.sum(y * y, axis=-1)])
    sq_col = sqn[:, :, None]
    sq_row = sqn[:, None, :]

    tasks = jnp.asarray(_build_tasks(nt))
    num_tasks = tasks.shape[1]
    tasks_per_core = num_tasks // 2

    def _a_map(c, s, T):
        t = c * tasks_per_core + s
        return T[0, t], T[1, t], 0

    def _b_map(c, s, T):
        t = c * tasks_per_core + s
        return T[2, t], T[3, t], 0

    def _a2_map(c, s, T):
        t = c * tasks_per_core + s
        return T[0, t], T[1, t], 0

    def _b2_map(c, s, T):
        t = c * tasks_per_core + s
        return T[2, t], 0, T[3, t]

    body = functools.partial(_mmd_task_kernel, tasks_per_core=tasks_per_core,
                             tn=tn, n_valid=n, needs_mask=needs_mask)
    sums = pl.pallas_call(
        body,
        out_shape=jax.ShapeDtypeStruct((2, 1, 1), jnp.float32),
        grid_spec=pltpu.PrefetchScalarGridSpec(
            num_scalar_prefetch=1,
            grid=(2, tasks_per_core),
            in_specs=[
                pl.BlockSpec((None, tn, 3 * d), _a_map),
                pl.BlockSpec((None, tn, 3 * d), _b_map),
                pl.BlockSpec((None, tn, 1), _a2_map),
                pl.BlockSpec((None, 1, tn), _b2_map),
            ],
            out_specs=pl.BlockSpec((None, 1, 1), lambda c, s, T: (c, 0, 0)),
        ),
        compiler_params=pltpu.CompilerParams(
            dimension_semantics=("parallel", "arbitrary")),
    )(tasks, a_op, b_op, sq_col, sq_row)

    return (sums[0, 0, 0] + sums[1, 0, 0]) * scale


def kernel(x, y):
    x = jnp.asarray(x, jnp.float32)
    y = jnp.asarray(y, jnp.float32)
    return _mmd_gaussian(x, y)


# speedup vs baseline: 13.0135x; 1.9321x over previous
import functools

import numpy as np

import jax
import jax.numpy as jnp
from jax import lax
from jax.experimental import pallas as pl
from jax.experimental.pallas import tpu as pltpu
from jax.experimental.shard_map import shard_map
from jax.sharding import Mesh, PartitionSpec as P

_NUM_GAMMAS = 19
_LOG2E = 1.4426950408889634


def _exp_g(m, g):
    return jnp.exp2(m * jnp.float32(g * _LOG2E))


def _kernel_sum(m):
    acc = _exp_g(m, 1e-6)
    p2 = acc * acc
    p4 = p2 * p2
    p8 = p4 * p4
    acc = acc + p8 * p2
    for g in (1e-4, 1e-3, 1e-2, 1e-1, 1.0):
        acc = acc + _exp_g(m, g)
    e5 = _exp_g(m, 5.0)
    h = 1.0 + e5
    for _ in range(5):
        h = 1.0 + e5 * h
    acc = acc + e5 * h
    for g in (100.0, 1000.0, 1e4, 1e5, 1e6):
        acc = acc + _exp_g(m, g)
    return acc


def _mmd_task_kernel(tasks_ref, a_ref, b_ref, a2_ref, b2_ref, o_ref, *,
                     tn, n_valid, needs_mask):
    t = pl.program_id(0)

    sel_a = tasks_ref[0, t]
    ti = tasks_ref[1, t]
    sel_b = tasks_ref[2, t]
    tj = tasks_ref[3, t]
    w = tasks_ref[4, t]

    @pl.when(t == 0)
    def _init():
        o_ref[...] = jnp.zeros_like(o_ref)

    a = a_ref[...]
    b = b_ref[...]
    ab2 = lax.dot_general(a, b,
                          dimension_numbers=(((1,), (1,)), ((), ())),
                          preferred_element_type=jnp.float32)
    ssum = a2_ref[...] + b2_ref[...]
    m = jnp.minimum(ab2 - ssum, 0.0)

    is_diag = jnp.logical_and(sel_a == sel_b, ti == tj)
    rows = lax.broadcasted_iota(jnp.int32, (tn, tn), 0)
    cols = lax.broadcasted_iota(jnp.int32, (tn, tn), 1)
    m = jnp.where(jnp.logical_and(is_diag, rows == cols), jnp.float32(0.0), m)

    k = _kernel_sum(m)

    if needs_mask:
        valid = jnp.logical_and(ti * tn + rows < n_valid,
                                tj * tn + cols < n_valid)
        k = jnp.where(valid, k, jnp.float32(0.0))

    o_ref[...] += w.astype(jnp.float32) * jnp.sum(k)


def _build_tasks(nt, shards):
    rows = []
    for sel_a, sel_b, self_k in ((0, 0, True), (1, 1, True), (0, 1, False)):
        for i in range(nt):
            for j in range(i if self_k else 0, nt):
                if self_k:
                    w = 1 if i == j else 2
                else:
                    w = -2
                rows.append((sel_a, i, sel_b, j, w))
    while len(rows) % shards:
        rows.append((0, 0, 1, 0, 0))
    arr = np.asarray(rows, dtype=np.int32)
    per = len(rows) // shards
    return arr.reshape(shards, per, 5).transpose(0, 2, 1)


def _mmd_gaussian(x, y):
    n, d = x.shape
    scale = jnp.float32(1.0 / (_NUM_GAMMAS * n * n))

    tn = next(t for t in (1024, 512, 256, 128, 64, 32, 16, 8)
              if n % t == 0 or t == 8)
    n_pad = ((n + tn - 1) // tn) * tn
    needs_mask = n_pad != n
    if needs_mask:
        x = jnp.pad(x, ((0, n_pad - n), (0, 0)))
        y = jnp.pad(y, ((0, n_pad - n), (0, 0)))
    nt = n_pad // tn

    xh = x.astype(jnp.bfloat16)
    xl = (x - xh.astype(jnp.float32)).astype(jnp.bfloat16)
    yh = y.astype(jnp.bfloat16)
    yl = (y - yh.astype(jnp.float32)).astype(jnp.bfloat16)
    a_op = jnp.stack([jnp.concatenate([xh, xh, xl], axis=1),
                      jnp.concatenate([yh, yh, yl], axis=1)])
    b_op = 2.0 * jnp.stack([jnp.concatenate([xh, xl, xh], axis=1),
                            jnp.concatenate([yh, yl, yh], axis=1)])
    sqn = jnp.stack([jnp.sum(x * x, axis=-1), jnp.sum(y * y, axis=-1)])
    sq_col = sqn[:, :, None]
    sq_row = sqn[:, None, :]

    devs = jax.devices()
    ndev = 2 if len(devs) >= 2 else 1
    tasks = jnp.asarray(_build_tasks(nt, ndev))
    per_shard = tasks.shape[2]

    body = functools.partial(_mmd_task_kernel, tn=tn, n_valid=n,
                             needs_mask=needs_mask)

    def _a_map(s, T):
        return T[0, s], T[1, s], 0

    def _b_map(s, T):
        return T[2, s], T[3, s], 0

    def _a2_map(s, T):
        return T[0, s], T[1, s], 0

    def _b2_map(s, T):
        return T[2, s], 0, T[3, s]

    def _run_shard(tk, a_o, b_o, sqc, sqr):
        return pl.pallas_call(
            body,
            out_shape=jax.ShapeDtypeStruct((1, 1), jnp.float32),
            grid_spec=pltpu.PrefetchScalarGridSpec(
                num_scalar_prefetch=1,
                grid=(per_shard,),
                in_specs=[
                    pl.BlockSpec((None, tn, 3 * d), _a_map),
                    pl.BlockSpec((None, tn, 3 * d), _b_map),
                    pl.BlockSpec((None, tn, 1), _a2_map),
                    pl.BlockSpec((None, 1, tn), _b2_map),
                ],
                out_specs=pl.BlockSpec((1, 1), lambda s, T: (0, 0)),
            ),
            compiler_params=pltpu.CompilerParams(
                dimension_semantics=("arbitrary",)),
        )(tk[0], a_o, b_o, sqc, sqr)

    if ndev == 1:
        sums = _run_shard(tasks, a_op, b_op, sq_col, sq_row)
        return (sums[0, 0]) * scale

    mesh = Mesh(np.array(devs[:ndev]), ("dp",))
    try:
        smap = shard_map(_run_shard, mesh=mesh,
                         in_specs=(P("dp"), P(), P(), P(), P()),
                         out_specs=P("dp"), check_vma=False)
    except TypeError:
        smap = shard_map(_run_shard, mesh=mesh,
                         in_specs=(P("dp"), P(), P(), P(), P()),
                         out_specs=P("dp"), check_rep=False)
    sums = smap(tasks, a_op, b_op, sq_col, sq_row)
    return jnp.sum(sums) * scale


def kernel(x, y):
    x = jnp.asarray(x, jnp.float32)
    y = jnp.asarray(y, jnp.float32)
    return _mmd_gaussian(x, y)
